# Initial kernel scaffold; baseline (speedup 1.0000x reference)
import jax
import jax.numpy as jnp
from jax.experimental import pallas as pl


def kernel(x, W, att_src, att_dst, bias, Wp, bp):
    raise NotImplementedError("write your pallas kernel here")


# speedup vs baseline: 278.1501x; 278.1501x over previous
import jax
import jax.numpy as jnp
from jax.experimental import pallas as pl

_B = 16
_C = 256
_F = 250
_H = 4
_O = 250
_K = 8
_NEG = float("-inf")


def _eeg_gat_kernel(xf_ref, wh_ref, att_s_ref, att_d_ref, bias_ref, wph_ref,
                    bp_ref, out_ref):
    f32 = jnp.float32
    xf = xf_ref[...]
    x0 = xf[0:_C, :]

    acc = jnp.zeros((_C, _C), f32)
    for b in range(_B):
        xb = xf[b * _C:(b + 1) * _C, :]
        mu = jnp.mean(xb, axis=1, keepdims=True)
        xc = xb - mu
        var = jnp.sum(xc * xc, axis=1, keepdims=True) * (1.0 / (_F - 1))
        xn = xc / (jnp.sqrt(var) + 1e-8)
        acc = acc + jax.lax.dot_general(
            xn, xn, (((1,), (1,)), ((), ())), preferred_element_type=f32)
    adj = acc * (1.0 / (_B * _F))

    work = adj
    thr = jnp.max(work, axis=1, keepdims=True)
    for _ in range(_K - 1):
        work = jnp.where(work < thr, work, _NEG)
        thr = jnp.max(work, axis=1, keepdims=True)
    mask = jnp.logical_and(adj >= thr, adj != 0.0)

    rid = jax.lax.broadcasted_iota(jnp.int32, (_C, _C), 0)
    cid = jax.lax.broadcasted_iota(jnp.int32, (_C, _C), 1)
    eye = rid == cid

    final0 = jnp.broadcast_to(bp_ref[...], (_C, _O)).astype(f32)
    wc = jnp.zeros((_O, _F), f32)
    bvec = jnp.zeros((1, _O), f32)
    for hd in range(_H):
        wh = wh_ref[hd]
        wph = wph_ref[hd]
        h0h = jax.lax.dot_general(
            x0, wh, (((1,), (1,)), ((), ())), preferred_element_type=f32)
        asc = jax.lax.dot_general(
            h0h, att_s_ref[hd:hd + 1, :], (((1,), (1,)), ((), ())),
            preferred_element_type=f32)
        adt = jax.lax.dot_general(
            att_d_ref[hd:hd + 1, :], h0h, (((1,), (1,)), ((), ())),
            preferred_element_type=f32)
        logit = asc + adt
        logit = jnp.where(logit > 0, logit, 0.2 * logit)
        lmask = jnp.where(mask, logit, _NEG)
        ldiag = jnp.max(jnp.where(eye, logit, _NEG), axis=0, keepdims=True)
        m = jnp.maximum(jnp.max(lmask, axis=0, keepdims=True), ldiag)
        e = jnp.exp(lmask - m)
        es = jnp.exp(ldiag - m)
        denom = jnp.sum(e, axis=0, keepdims=True) + es
        attw = (e + jnp.where(eye, es, 0.0)) / denom
        attn = jax.lax.dot_general(
            attw, h0h, (((0,), (0,)), ((), ())), preferred_element_type=f32)
        final0 = final0 + jax.lax.dot_general(
            attn + bias_ref[hd:hd + 1, :], wph, (((1,), (1,)), ((), ())),
            preferred_element_type=f32)
        wc = wc + jax.lax.dot_general(
            wph, wh, (((1,), (0,)), ((), ())), preferred_element_type=f32)
        bvec = bvec + jax.lax.dot_general(
            bias_ref[hd:hd + 1, :], wph, (((1,), (1,)), ((), ())),
            preferred_element_type=f32)

    out_ref[...] = jax.lax.dot_general(
        xf, wc, (((1,), (1,)), ((), ())),
        preferred_element_type=f32) + bvec + bp_ref[...]
    out_ref[0:_C, :] = final0


def kernel(x, W, att_src, att_dst, bias, Wp, bp):
    xf = x.reshape(_B * _C, _F)
    wh = W.reshape(_H, _O, _F)
    wph = Wp.reshape(_O, _H, _O).transpose(1, 0, 2)
    att_s = att_src.reshape(_H, _O)
    att_d = att_dst.reshape(_H, _O)
    bias_h = bias.reshape(_H, _O)
    bp2 = bp.reshape(1, _O)
    out = pl.pallas_call(
        _eeg_gat_kernel,
        out_shape=jax.ShapeDtypeStruct((_B * _C, _F), jnp.float32),
    )(xf, wh, att_s, att_d, bias_h, wph, bp2)
    return out.reshape(_B, 1, _C, _O)
